# Initial kernel scaffold; baseline (speedup 1.0000x reference)
import jax
import jax.numpy as jnp
from jax.experimental import pallas as pl


def kernel(x, edge_index, adj_values, W1, W2):
    raise NotImplementedError("write your pallas kernel here")


# speedup vs baseline: 15.0364x; 15.0364x over previous
import dataclasses
import functools

import jax
import jax.numpy as jnp
from jax import lax
from jax.experimental import pallas as pl
from jax.experimental.pallas import tpu as pltpu
from jax.experimental.pallas import tpu_sc as plsc

NC = 2
NS = 16
LANES = 16
CHUNK = 128


def _row_block(n):
    for blk in (2000, 1000, 500, 200, 100, 50, 25, 10, 8):
        if n % blk == 0:
            return blk
    return n


def _gates(x, W1, W2):
    n, d = x.shape
    blk = _row_block(n)

    def body(x_ref, w1_ref, w2_ref, o1_ref, o2_ref):
        xb = x_ref[...]
        o1_ref[...] = jnp.sum(xb * w1_ref[...], axis=1, keepdims=True)
        o2_ref[...] = jnp.sum(xb * w2_ref[...], axis=1, keepdims=True)

    o1, o2 = pl.pallas_call(
        body,
        grid=(n // blk,),
        in_specs=[
            pl.BlockSpec((blk, d), lambda i: (i, 0)),
            pl.BlockSpec((1, d), lambda i: (0, 0)),
            pl.BlockSpec((1, d), lambda i: (0, 0)),
        ],
        out_specs=[
            pl.BlockSpec((blk, 1), lambda i: (i, 0)),
            pl.BlockSpec((blk, 1), lambda i: (i, 0)),
        ],
        out_shape=[
            jax.ShapeDtypeStruct((n, 1), jnp.float32),
            jax.ShapeDtypeStruct((n, 1), jnp.float32),
        ],
    )(x, W1, W2)
    return o1.reshape(n), o2.reshape(n)


def _sum_partials(p):
    _, n, d = p.shape
    blk = _row_block(n)

    def body(p_ref, o_ref):
        o_ref[...] = p_ref[0] + p_ref[1]

    return pl.pallas_call(
        body,
        grid=(n // blk,),
        in_specs=[pl.BlockSpec((2, blk, d), lambda i: (0, i, 0))],
        out_specs=pl.BlockSpec((blk, d), lambda i: (i, 0)),
        out_shape=jax.ShapeDtypeStruct((n, d), jnp.float32),
    )(p)


def _sc_aggregate(x, src, dst, adj, x1, x2):
    n, d = x.shape
    epad = src.shape[0]
    epw = epad // (NC * NS)
    nchunks = epw // CHUNK
    zrows = (n // NS) // 8 * 8
    tail = n - zrows * NS
    zsizes = []
    left = zrows
    while left > 0:
        blk = min(left, CHUNK)
        zsizes.append(blk)
        left -= blk

    mesh = plsc.VectorSubcoreMesh(core_axis_name="c", subcore_axis_name="s")
    cp = pltpu.CompilerParams()
    if "needs_layout_passes" in pltpu.CompilerParams.__dataclass_fields__:
        cp = dataclasses.replace(cp, needs_layout_passes=False)

    @functools.partial(
        pl.kernel,
        out_type=jax.ShapeDtypeStruct((NC, n, d), jnp.float32),
        mesh=mesh,
        compiler_params=cp,
        scratch_types=[
            pltpu.VMEM((n,), jnp.float32),
            pltpu.VMEM((n,), jnp.float32),
            pltpu.VMEM((CHUNK,), jnp.int32),
            pltpu.VMEM((CHUNK,), jnp.int32),
            pltpu.VMEM((CHUNK,), jnp.float32),
            pltpu.VMEM((CHUNK,), jnp.float32),
            pltpu.VMEM((CHUNK, d), jnp.float32),
            pltpu.VMEM_SHARED((n, d), jnp.float32),
        ],
    )
    def sc_kernel(x_hbm, src_hbm, dst_hbm, adj_hbm, x1_hbm, x2_hbm, out_hbm,
                  x1t, x2t, tsrc, tdst, tadj, tm, rows, accum):
        c = lax.axis_index("c")
        s = lax.axis_index("s")

        pltpu.sync_copy(x1_hbm, x1t)
        pltpu.sync_copy(x2_hbm, x2t)

        @pl.loop(0, CHUNK)
        def _zero_rows(i):
            for j in range(d // LANES):
                rows[i, pl.ds(j * LANES, LANES)] = jnp.zeros((LANES,), jnp.float32)

        off = 0
        for blk in zsizes:
            pltpu.sync_copy(rows.at[pl.ds(0, blk)],
                            accum.at[pl.ds(s * zrows + off, blk)])
            off += blk
        if tail:
            @pl.when(s == NS - 1)
            def _zero_tail():
                pltpu.sync_copy(rows.at[pl.ds(0, tail)],
                                accum.at[pl.ds(NS * zrows, tail)])

        plsc.subcore_barrier()

        base0 = (c * NS + s) * epw

        @pl.loop(0, nchunks)
        def _edge_chunk(k):
            base = base0 + k * CHUNK
            pltpu.sync_copy(src_hbm.at[pl.ds(base, CHUNK)], tsrc)
            pltpu.sync_copy(dst_hbm.at[pl.ds(base, CHUNK)], tdst)
            pltpu.sync_copy(adj_hbm.at[pl.ds(base, CHUNK)], tadj)
            pltpu.sync_copy(x_hbm.at[tdst], rows)

            @pl.loop(0, CHUNK, step=LANES)
            def _gate(j):
                idxs = tsrc[pl.ds(j, LANES)]
                idxd = tdst[pl.ds(j, LANES)]
                s1 = plsc.load_gather(x1t, [idxs])
                s2 = plsc.load_gather(x2t, [idxd])
                e2 = jnp.exp((s1 + s2) * 2.0)
                tm[pl.ds(j, LANES)] = (1.0 - 2.0 / (e2 + 1.0)) * tadj[pl.ds(j, LANES)]

            @pl.loop(0, CHUNK, step=LANES)
            def _scale(g):
                mv = tm[pl.ds(g, LANES)]
                for i in range(LANES):
                    mi = mv[i]
                    for j in range(d // LANES):
                        sl = pl.ds(j * LANES, LANES)
                        rows[g + i, sl] = rows[g + i, sl] * mi

            pltpu.sync_copy(rows, accum.at[tsrc], add=True)

        plsc.subcore_barrier()

        r0 = s * zrows
        pltpu.sync_copy(accum.at[pl.ds(r0, zrows)], out_hbm.at[c, pl.ds(r0, zrows)])
        if tail:
            @pl.when(s == NS - 1)
            def _write_tail():
                pltpu.sync_copy(accum.at[pl.ds(NS * zrows, tail)],
                                out_hbm.at[c, pl.ds(NS * zrows, tail)])

    return sc_kernel(x, src, dst, adj, x1, x2)


def kernel(x, edge_index, adj_values, W1, W2):
    n, d = x.shape
    e = edge_index.shape[1]

    x1, x2 = _gates(x, W1, W2)

    quantum = NC * NS * CHUNK
    epad = ((e + quantum - 1) // quantum) * quantum
    pad = epad - e
    src = jnp.concatenate([edge_index[0], jnp.zeros((pad,), jnp.int32)])
    dst = jnp.concatenate([edge_index[1], jnp.zeros((pad,), jnp.int32)])
    adj = jnp.concatenate([adj_values, jnp.zeros((pad,), jnp.float32)])

    partials = _sc_aggregate(x, src, dst, adj, x1, x2)
    return _sum_partials(partials)
